# Initial kernel scaffold; baseline (speedup 1.0000x reference)
import functools

import jax
import jax.numpy as jnp
from jax import lax
from jax.experimental import pallas as pl
from jax.experimental.pallas import tpu as pltpu
from jax.experimental.pallas import tpu_sc as plsc

_N = 10000
_E = 320000
_D = 128
_NC = 2
_NS = 16
_NW = _NC * _NS
_EPW = _E // _NW
_CH = 80
_NCHUNK = _EPW // _CH
_RPT = _N // _NS
_DEGW = 16

_mesh = plsc.VectorSubcoreMesh(core_axis_name="c", subcore_axis_name="s")


@functools.partial(
    pl.kernel,
    mesh=_mesh,
    out_type=jax.ShapeDtypeStruct((_NC, _N, _DEGW), jnp.float32),
    scratch_types=[
        pltpu.VMEM_SHARED((_N, _DEGW), jnp.float32),
        pltpu.VMEM((_CH,), jnp.int32),
        pltpu.VMEM((_CH, _DEGW), jnp.float32),
    ],
)
def _deg_kernel(dst_hbm, ones_hbm, zer_hbm, out_hbm, deg_sh, dstv, onesv):
    cid = lax.axis_index("c")
    sid = lax.axis_index("s")
    wid = sid * _NC + cid
    pltpu.sync_copy(zer_hbm, deg_sh.at[pl.ds(sid * _RPT, _RPT)])
    pltpu.sync_copy(ones_hbm, onesv)
    plsc.subcore_barrier()

    def chunk(c, carry):
        base = wid * _EPW + c * _CH
        pltpu.sync_copy(dst_hbm.at[pl.ds(base, _CH)], dstv)
        pltpu.sync_copy(onesv, deg_sh.at[dstv], add=True)
        return carry

    lax.fori_loop(0, _NCHUNK, chunk, 0)
    plsc.subcore_barrier()
    pltpu.sync_copy(deg_sh.at[pl.ds(sid * _RPT, _RPT)],
                    out_hbm.at[cid, pl.ds(sid * _RPT, _RPT)])


@functools.partial(
    pl.kernel,
    mesh=_mesh,
    out_type=jax.ShapeDtypeStruct((_NC, _N, _D), jnp.float32),
    scratch_types=[
        pltpu.VMEM_SHARED((_N, _D), jnp.float32),
        pltpu.VMEM((_CH,), jnp.int32),
        pltpu.VMEM((_CH,), jnp.int32),
        pltpu.VMEM((_CH, _D), jnp.float32),
        pltpu.SemaphoreType.DMA,
    ],
)
def _scat_kernel(y_hbm, src_hbm, dst_hbm, zer_hbm, out_hbm,
                 acc_sh, srcv, dstv, rows, sem):
    cid = lax.axis_index("c")
    sid = lax.axis_index("s")
    wid = sid * _NC + cid
    pltpu.sync_copy(zer_hbm, acc_sh.at[pl.ds(sid * _RPT, _RPT)])
    plsc.subcore_barrier()

    def chunk(c, carry):
        base = wid * _EPW + c * _CH
        pltpu.sync_copy(src_hbm.at[pl.ds(base, _CH)], srcv)
        pltpu.sync_copy(dst_hbm.at[pl.ds(base, _CH)], dstv)
        pltpu.async_copy(y_hbm.at[srcv], rows, sem).wait()
        pltpu.sync_copy(rows, acc_sh.at[dstv], add=True)
        return carry

    lax.fori_loop(0, _NCHUNK, chunk, 0)
    plsc.subcore_barrier()
    pltpu.sync_copy(acc_sh.at[pl.ds(sid * _RPT, _RPT)],
                    out_hbm.at[cid, pl.ds(sid * _RPT, _RPT)])


_BLK = 1000
_NBLK = _N // _BLK


def _leaky(v):
    return jnp.where(v > 0, v, 0.01 * v)


def _tc1_body(pose_ref, degp_ref, w1_ref, b1_ref, w2_ref, b2_ref, wg1_ref,
              y1_ref, dinv_ref):
    deg = degp_ref[0, :, 0:1] + degp_ref[1, :, 0:1] + 1.0
    dinv = lax.rsqrt(deg)
    x = jnp.dot(pose_ref[...], w1_ref[...],
                preferred_element_type=jnp.float32) + b1_ref[...]
    x = _leaky(x)
    x = jnp.dot(x, w2_ref[...],
                preferred_element_type=jnp.float32) + b2_ref[...]
    y1_ref[...] = dinv * jnp.dot(x, wg1_ref[...],
                                 preferred_element_type=jnp.float32)
    dinv_ref[...] = lax.broadcast_in_dim(dinv, (_BLK, 8), (0, 1))


_tc1 = pl.pallas_call(
    _tc1_body,
    grid=(_NBLK,),
    in_specs=[
        pl.BlockSpec((_BLK, _D), lambda i: (i, 0)),
        pl.BlockSpec((_NC, _BLK, _DEGW), lambda i: (0, i, 0)),
        pl.BlockSpec((_D, _D), lambda i: (0, 0)),
        pl.BlockSpec((1, _D), lambda i: (0, 0)),
        pl.BlockSpec((_D, _D), lambda i: (0, 0)),
        pl.BlockSpec((1, _D), lambda i: (0, 0)),
        pl.BlockSpec((_D, _D), lambda i: (0, 0)),
    ],
    out_specs=[
        pl.BlockSpec((_BLK, _D), lambda i: (i, 0)),
        pl.BlockSpec((_BLK, 8), lambda i: (i, 0)),
    ],
    out_shape=[
        jax.ShapeDtypeStruct((_N, _D), jnp.float32),
        jax.ShapeDtypeStruct((_N, 8), jnp.float32),
    ],
)


def _tc2_body(accp_ref, y1_ref, dinv_ref, bg1_ref, wg2_ref, y2_ref):
    dinv = dinv_ref[:, 0:1]
    a = accp_ref[0] + accp_ref[1] + y1_ref[...]
    h = _leaky(dinv * a + bg1_ref[...])
    y2_ref[...] = dinv * jnp.dot(h, wg2_ref[...],
                                 preferred_element_type=jnp.float32)


_tc2 = pl.pallas_call(
    _tc2_body,
    grid=(_NBLK,),
    in_specs=[
        pl.BlockSpec((_NC, _BLK, _D), lambda i: (0, i, 0)),
        pl.BlockSpec((_BLK, _D), lambda i: (i, 0)),
        pl.BlockSpec((_BLK, 8), lambda i: (i, 0)),
        pl.BlockSpec((1, _D), lambda i: (0, 0)),
        pl.BlockSpec((_D, _D), lambda i: (0, 0)),
    ],
    out_specs=pl.BlockSpec((_BLK, _D), lambda i: (i, 0)),
    out_shape=jax.ShapeDtypeStruct((_N, _D), jnp.float32),
)


def _tc3_body(accp_ref, y2_ref, dinv_ref, bg2_ref, out_ref):
    i = pl.program_id(0)
    dinv = dinv_ref[:, 0:1]
    a = accp_ref[0] + accp_ref[1] + y2_ref[...]
    h = _leaky(dinv * a + bg2_ref[...])
    s = jnp.sum(h, axis=0, keepdims=True) * (1.0 / _N)

    @pl.when(i == 0)
    def _():
        out_ref[...] = s

    @pl.when(i != 0)
    def _():
        out_ref[...] = out_ref[...] + s


_tc3 = pl.pallas_call(
    _tc3_body,
    grid=(_NBLK,),
    in_specs=[
        pl.BlockSpec((_NC, _BLK, _D), lambda i: (0, i, 0)),
        pl.BlockSpec((_BLK, _D), lambda i: (i, 0)),
        pl.BlockSpec((_BLK, 8), lambda i: (i, 0)),
        pl.BlockSpec((1, _D), lambda i: (0, 0)),
    ],
    out_specs=pl.BlockSpec((1, _D), lambda i: (0, 0)),
    out_shape=jax.ShapeDtypeStruct((1, _D), jnp.float32),
)


def kernel(pose, edge_index, W1, b1, W2, b2, Wg1, bg1, Wg2, bg2):
    src = edge_index[0]
    dst = edge_index[1]
    b1r = b1.reshape(1, _D)
    b2r = b2.reshape(1, _D)
    bg1r = bg1.reshape(1, _D)
    bg2r = bg2.reshape(1, _D)
    ones_deg = jnp.ones((_CH, _DEGW), jnp.float32)
    zer_deg = jnp.zeros((_RPT, _DEGW), jnp.float32)
    zer_row = jnp.zeros((_RPT, _D), jnp.float32)

    degp = _deg_kernel(dst, ones_deg, zer_deg)
    y1, dinv8 = _tc1(pose, degp, W1, b1r, W2, b2r, Wg1)
    accp1 = _scat_kernel(y1, src, dst, zer_row)
    y2 = _tc2(accp1, y1, dinv8, bg1r, Wg2)
    accp2 = _scat_kernel(y2, src, dst, zer_row)
    g = _tc3(accp2, y2, dinv8, bg2r)
    return g


# speedup vs baseline: 12.4846x; 12.4846x over previous
import functools

import jax
import jax.numpy as jnp
from jax import lax
from jax.experimental import pallas as pl
from jax.experimental.pallas import tpu as pltpu
from jax.experimental.pallas import tpu_sc as plsc

_N = 10000
_E = 320000
_D = 128
_NC = 2
_NS = 16
_NW = _NC * _NS
_EPW = _E // _NW
_CH = 80
_NCHUNK = _EPW // _CH
_NP = 10240
_RPT = _NP // _NS
_DEGW = 128

_mesh = plsc.VectorSubcoreMesh(core_axis_name="c", subcore_axis_name="s")


@functools.partial(
    pl.kernel,
    mesh=_mesh,
    out_type=jax.ShapeDtypeStruct((_NC, _NP, _DEGW), jnp.float32),
    scratch_types=[
        pltpu.VMEM_SHARED((_NP, _DEGW), jnp.float32),
        pltpu.VMEM((_CH,), jnp.int32),
        pltpu.VMEM((_CH, _DEGW), jnp.float32),
    ],
)
def _deg_kernel(dst_hbm, ones_hbm, zer_hbm, out_hbm, deg_sh, dstv, onesv):
    cid = lax.axis_index("c")
    sid = lax.axis_index("s")
    wid = sid * _NC + cid
    pltpu.sync_copy(zer_hbm, deg_sh.at[pl.ds(sid * _RPT, _RPT)])
    pltpu.sync_copy(ones_hbm, onesv)
    plsc.subcore_barrier()

    def chunk(c, carry):
        base = wid * _EPW + c * _CH
        pltpu.sync_copy(dst_hbm.at[pl.ds(base, _CH)], dstv)
        pltpu.sync_copy(onesv, deg_sh.at[dstv], add=True)
        return carry

    lax.fori_loop(0, _NCHUNK, chunk, 0)
    plsc.subcore_barrier()
    pltpu.sync_copy(deg_sh.at[pl.ds(sid * _RPT, _RPT)],
                    out_hbm.at[cid, pl.ds(sid * _RPT, _RPT)])


@functools.partial(
    pl.kernel,
    mesh=_mesh,
    out_type=jax.ShapeDtypeStruct((_NC, _NP, _D), jnp.float32),
    scratch_types=[
        pltpu.VMEM_SHARED((_NP, _D), jnp.float32),
        pltpu.VMEM((_CH,), jnp.int32),
        pltpu.VMEM((_CH,), jnp.int32),
        pltpu.VMEM((_CH, _D), jnp.float32),
        pltpu.SemaphoreType.DMA,
    ],
)
def _scat_kernel(y_hbm, src_hbm, dst_hbm, zer_hbm, out_hbm,
                 acc_sh, srcv, dstv, rows, sem):
    cid = lax.axis_index("c")
    sid = lax.axis_index("s")
    wid = sid * _NC + cid
    pltpu.sync_copy(zer_hbm, acc_sh.at[pl.ds(sid * _RPT, _RPT)])
    plsc.subcore_barrier()

    def chunk(c, carry):
        base = wid * _EPW + c * _CH
        pltpu.sync_copy(src_hbm.at[pl.ds(base, _CH)], srcv)
        pltpu.sync_copy(dst_hbm.at[pl.ds(base, _CH)], dstv)
        pltpu.async_copy(y_hbm.at[srcv], rows, sem).wait()
        pltpu.sync_copy(rows, acc_sh.at[dstv], add=True)
        return carry

    lax.fori_loop(0, _NCHUNK, chunk, 0)
    plsc.subcore_barrier()
    pltpu.sync_copy(acc_sh.at[pl.ds(sid * _RPT, _RPT)],
                    out_hbm.at[cid, pl.ds(sid * _RPT, _RPT)])


_BLK = 1000
_NBLK = _N // _BLK


def _leaky(v):
    return jnp.where(v > 0, v, 0.01 * v)


def _tc1_body(pose_ref, degp_ref, w1_ref, b1_ref, w2_ref, b2_ref, wg1_ref,
              y1_ref, dinv_ref):
    deg = degp_ref[0, :, 0:1] + degp_ref[1, :, 0:1] + 1.0
    dinv = lax.rsqrt(deg)
    x = jnp.dot(pose_ref[...], w1_ref[...],
                preferred_element_type=jnp.float32) + b1_ref[...]
    x = _leaky(x)
    x = jnp.dot(x, w2_ref[...],
                preferred_element_type=jnp.float32) + b2_ref[...]
    y1_ref[...] = dinv * jnp.dot(x, wg1_ref[...],
                                 preferred_element_type=jnp.float32)
    dinv_ref[...] = lax.broadcast_in_dim(dinv, (_BLK, 8), (0, 1))


_tc1 = pl.pallas_call(
    _tc1_body,
    grid=(_NBLK,),
    in_specs=[
        pl.BlockSpec((_BLK, _D), lambda i: (i, 0)),
        pl.BlockSpec((_NC, _BLK, _DEGW), lambda i: (0, i, 0)),
        pl.BlockSpec((_D, _D), lambda i: (0, 0)),
        pl.BlockSpec((1, _D), lambda i: (0, 0)),
        pl.BlockSpec((_D, _D), lambda i: (0, 0)),
        pl.BlockSpec((1, _D), lambda i: (0, 0)),
        pl.BlockSpec((_D, _D), lambda i: (0, 0)),
    ],
    out_specs=[
        pl.BlockSpec((_BLK, _D), lambda i: (i, 0)),
        pl.BlockSpec((_BLK, 8), lambda i: (i, 0)),
    ],
    out_shape=[
        jax.ShapeDtypeStruct((_N, _D), jnp.float32),
        jax.ShapeDtypeStruct((_N, 8), jnp.float32),
    ],
)


def _tc2_body(accp_ref, y1_ref, dinv_ref, bg1_ref, wg2_ref, y2_ref):
    dinv = dinv_ref[:, 0:1]
    a = accp_ref[0] + accp_ref[1] + y1_ref[...]
    h = _leaky(dinv * a + bg1_ref[...])
    y2_ref[...] = dinv * jnp.dot(h, wg2_ref[...],
                                 preferred_element_type=jnp.float32)


_tc2 = pl.pallas_call(
    _tc2_body,
    grid=(_NBLK,),
    in_specs=[
        pl.BlockSpec((_NC, _BLK, _D), lambda i: (0, i, 0)),
        pl.BlockSpec((_BLK, _D), lambda i: (i, 0)),
        pl.BlockSpec((_BLK, 8), lambda i: (i, 0)),
        pl.BlockSpec((1, _D), lambda i: (0, 0)),
        pl.BlockSpec((_D, _D), lambda i: (0, 0)),
    ],
    out_specs=pl.BlockSpec((_BLK, _D), lambda i: (i, 0)),
    out_shape=jax.ShapeDtypeStruct((_N, _D), jnp.float32),
)


def _tc3_body(accp_ref, y2_ref, dinv_ref, bg2_ref, out_ref):
    i = pl.program_id(0)
    dinv = dinv_ref[:, 0:1]
    a = accp_ref[0] + accp_ref[1] + y2_ref[...]
    h = _leaky(dinv * a + bg2_ref[...])
    s = jnp.sum(h, axis=0, keepdims=True) * (1.0 / _N)

    @pl.when(i == 0)
    def _():
        out_ref[...] = s

    @pl.when(i != 0)
    def _():
        out_ref[...] = out_ref[...] + s


_tc3 = pl.pallas_call(
    _tc3_body,
    grid=(_NBLK,),
    in_specs=[
        pl.BlockSpec((_NC, _BLK, _D), lambda i: (0, i, 0)),
        pl.BlockSpec((_BLK, _D), lambda i: (i, 0)),
        pl.BlockSpec((_BLK, 8), lambda i: (i, 0)),
        pl.BlockSpec((1, _D), lambda i: (0, 0)),
    ],
    out_specs=pl.BlockSpec((1, _D), lambda i: (0, 0)),
    out_shape=jax.ShapeDtypeStruct((1, _D), jnp.float32),
)


def kernel(pose, edge_index, W1, b1, W2, b2, Wg1, bg1, Wg2, bg2):
    src = edge_index[0]
    dst = edge_index[1]
    b1r = b1.reshape(1, _D)
    b2r = b2.reshape(1, _D)
    bg1r = bg1.reshape(1, _D)
    bg2r = bg2.reshape(1, _D)
    ones_deg = jnp.ones((_CH, _DEGW), jnp.float32)
    zer_row = jnp.zeros((_RPT, _D), jnp.float32)

    degp = _deg_kernel(dst, ones_deg, zer_row)
    y1, dinv8 = _tc1(pose, degp, W1, b1r, W2, b2r, Wg1)
    accp1 = _scat_kernel(y1, src, dst, zer_row)
    y2 = _tc2(accp1, y1, dinv8, bg1r, Wg2)
    accp2 = _scat_kernel(y2, src, dst, zer_row)
    g = _tc3(accp2, y2, dinv8, bg2r)
    return g


# speedup vs baseline: 22.2639x; 1.7833x over previous
import functools

import jax
import jax.numpy as jnp
from jax import lax
from jax.experimental import pallas as pl
from jax.experimental.pallas import tpu as pltpu
from jax.experimental.pallas import tpu_sc as plsc

_N = 10000
_E = 320000
_D = 128
_NC = 2
_NS = 16
_NW = _NC * _NS
_EPW = _E // _NW
_CH = 80
_NCHUNK = _EPW // _CH
_NP = 10240
_RPT = _NP // _NS

_mesh = plsc.VectorSubcoreMesh(core_axis_name="c", subcore_axis_name="s")


@functools.partial(
    pl.kernel,
    mesh=_mesh,
    out_type=jax.ShapeDtypeStruct((_NC, _NP, _D), jnp.float32),
    scratch_types=[
        pltpu.VMEM_SHARED((_NP, _D), jnp.float32),
        pltpu.VMEM((2, _CH), jnp.int32),
        pltpu.VMEM((2, _CH), jnp.int32),
        pltpu.VMEM((_CH, _D), jnp.float32),
        pltpu.SemaphoreType.DMA,
        pltpu.SemaphoreType.DMA,
    ],
)
def _deg_kernel(ep_hbm, ones_hbm, zer_hbm, out_hbm,
                deg_sh, idx0, idx1, onesv, sma, smb):
    cid = lax.axis_index("c")
    sid = lax.axis_index("s")
    wid = sid * _NC + cid
    pltpu.sync_copy(ones_hbm, onesv)
    pltpu.sync_copy(zer_hbm, deg_sh.at[pl.ds(sid * _RPT, _RPT)])
    plsc.subcore_barrier()
    pltpu.sync_copy(ep_hbm.at[wid, 0], idx0)
    pltpu.make_async_copy(ep_hbm.at[wid, 1], idx1, smb).start()

    def body(i, carry):
        a = 2 * i
        pltpu.sync_copy(onesv, deg_sh.at[idx0.at[1]], add=True)
        pltpu.make_async_copy(ep_hbm.at[wid, a + 2], idx0, sma).start()
        pltpu.make_async_copy(ep_hbm.at[wid, a + 1], idx1, smb).wait()
        pltpu.sync_copy(onesv, deg_sh.at[idx1.at[1]], add=True)
        pltpu.make_async_copy(ep_hbm.at[wid, a + 3], idx1, smb).start()
        pltpu.make_async_copy(ep_hbm.at[wid, a + 2], idx0, sma).wait()
        return carry

    lax.fori_loop(0, (_NCHUNK - 1) // 2, body, 0)
    pltpu.sync_copy(onesv, deg_sh.at[idx0.at[1]], add=True)
    pltpu.make_async_copy(ep_hbm.at[wid, _NCHUNK], idx1, smb).wait()
    plsc.subcore_barrier()
    pltpu.sync_copy(deg_sh.at[pl.ds(sid * _RPT, _RPT)],
                    out_hbm.at[cid, pl.ds(sid * _RPT, _RPT)])


@functools.partial(
    pl.kernel,
    mesh=_mesh,
    out_type=jax.ShapeDtypeStruct((_NC, _NP, _D), jnp.float32),
    scratch_types=[
        pltpu.VMEM_SHARED((_NP, _D), jnp.float32),
        pltpu.VMEM((2, _CH), jnp.int32),
        pltpu.VMEM((2, _CH), jnp.int32),
        pltpu.VMEM((_CH, _D), jnp.float32),
        pltpu.VMEM((_CH, _D), jnp.float32),
        pltpu.SemaphoreType.DMA,
        pltpu.SemaphoreType.DMA,
    ],
)
def _scat_kernel(y_hbm, ep_hbm, zer_hbm, out_hbm,
                 acc_sh, idx0, idx1, rows0, rows1, gsa, gsb):
    cid = lax.axis_index("c")
    sid = lax.axis_index("s")
    wid = sid * _NC + cid
    pltpu.sync_copy(zer_hbm, acc_sh.at[pl.ds(sid * _RPT, _RPT)])
    plsc.subcore_barrier()
    pltpu.sync_copy(ep_hbm.at[wid, 0], idx0)
    pltpu.make_async_copy(y_hbm.at[idx0.at[0]], rows0, gsa).start()
    pltpu.sync_copy(ep_hbm.at[wid, 1], idx1)

    def body(i, carry):
        a = 2 * i
        pltpu.make_async_copy(y_hbm.at[idx1.at[0]], rows1, gsb).start()
        pltpu.make_async_copy(y_hbm.at[idx0.at[0]], rows0, gsa).wait()
        pltpu.sync_copy(rows0, acc_sh.at[idx0.at[1]], add=True)
        pltpu.sync_copy(ep_hbm.at[wid, a + 2], idx0)
        pltpu.make_async_copy(y_hbm.at[idx0.at[0]], rows0, gsa).start()
        pltpu.make_async_copy(y_hbm.at[idx1.at[0]], rows1, gsb).wait()
        pltpu.sync_copy(rows1, acc_sh.at[idx1.at[1]], add=True)
        pltpu.sync_copy(ep_hbm.at[wid, a + 3], idx1)
        return carry

    lax.fori_loop(0, (_NCHUNK - 1) // 2, body, 0)
    pltpu.make_async_copy(y_hbm.at[idx0.at[0]], rows0, gsa).wait()
    pltpu.sync_copy(rows0, acc_sh.at[idx0.at[1]], add=True)
    plsc.subcore_barrier()
    pltpu.sync_copy(acc_sh.at[pl.ds(sid * _RPT, _RPT)],
                    out_hbm.at[cid, pl.ds(sid * _RPT, _RPT)])


_BLK = 1000
_NBLK = _N // _BLK


def _leaky(v):
    return jnp.where(v > 0, v, 0.01 * v)


def _tc1_body(pose_ref, degp_ref, w1_ref, b1_ref, w2_ref, b2_ref, wg1_ref,
              y1_ref, dinv_ref):
    deg = degp_ref[0, :, 0:1] + degp_ref[1, :, 0:1] + 1.0
    dinv = lax.rsqrt(deg)
    x = jnp.dot(pose_ref[...], w1_ref[...],
                preferred_element_type=jnp.float32) + b1_ref[...]
    x = _leaky(x)
    x = jnp.dot(x, w2_ref[...],
                preferred_element_type=jnp.float32) + b2_ref[...]
    y1_ref[...] = dinv * jnp.dot(x, wg1_ref[...],
                                 preferred_element_type=jnp.float32)
    dinv_ref[...] = lax.broadcast_in_dim(dinv, (_BLK, 8), (0, 1))


_tc1 = pl.pallas_call(
    _tc1_body,
    grid=(_NBLK,),
    in_specs=[
        pl.BlockSpec((_BLK, _D), lambda i: (i, 0)),
        pl.BlockSpec((_NC, _BLK, _D), lambda i: (0, i, 0)),
        pl.BlockSpec((_D, _D), lambda i: (0, 0)),
        pl.BlockSpec((1, _D), lambda i: (0, 0)),
        pl.BlockSpec((_D, _D), lambda i: (0, 0)),
        pl.BlockSpec((1, _D), lambda i: (0, 0)),
        pl.BlockSpec((_D, _D), lambda i: (0, 0)),
    ],
    out_specs=[
        pl.BlockSpec((_BLK, _D), lambda i: (i, 0)),
        pl.BlockSpec((_BLK, 8), lambda i: (i, 0)),
    ],
    out_shape=[
        jax.ShapeDtypeStruct((_N, _D), jnp.float32),
        jax.ShapeDtypeStruct((_N, 8), jnp.float32),
    ],
)


def _tc2_body(accp_ref, y1_ref, dinv_ref, bg1_ref, wg2_ref, y2_ref):
    dinv = dinv_ref[:, 0:1]
    a = accp_ref[0] + accp_ref[1] + y1_ref[...]
    h = _leaky(dinv * a + bg1_ref[...])
    y2_ref[...] = dinv * jnp.dot(h, wg2_ref[...],
                                 preferred_element_type=jnp.float32)


_tc2 = pl.pallas_call(
    _tc2_body,
    grid=(_NBLK,),
    in_specs=[
        pl.BlockSpec((_NC, _BLK, _D), lambda i: (0, i, 0)),
        pl.BlockSpec((_BLK, _D), lambda i: (i, 0)),
        pl.BlockSpec((_BLK, 8), lambda i: (i, 0)),
        pl.BlockSpec((1, _D), lambda i: (0, 0)),
        pl.BlockSpec((_D, _D), lambda i: (0, 0)),
    ],
    out_specs=pl.BlockSpec((_BLK, _D), lambda i: (i, 0)),
    out_shape=jax.ShapeDtypeStruct((_N, _D), jnp.float32),
)


def _tc3_body(accp_ref, y2_ref, dinv_ref, bg2_ref, out_ref):
    i = pl.program_id(0)
    dinv = dinv_ref[:, 0:1]
    a = accp_ref[0] + accp_ref[1] + y2_ref[...]
    h = _leaky(dinv * a + bg2_ref[...])
    s = jnp.sum(h, axis=0, keepdims=True) * (1.0 / _N)

    @pl.when(i == 0)
    def _():
        out_ref[...] = s

    @pl.when(i != 0)
    def _():
        out_ref[...] = out_ref[...] + s


_tc3 = pl.pallas_call(
    _tc3_body,
    grid=(_NBLK,),
    in_specs=[
        pl.BlockSpec((_NC, _BLK, _D), lambda i: (0, i, 0)),
        pl.BlockSpec((_BLK, _D), lambda i: (i, 0)),
        pl.BlockSpec((_BLK, 8), lambda i: (i, 0)),
        pl.BlockSpec((1, _D), lambda i: (0, 0)),
    ],
    out_specs=pl.BlockSpec((1, _D), lambda i: (0, 0)),
    out_shape=jax.ShapeDtypeStruct((1, _D), jnp.float32),
)


def kernel(pose, edge_index, W1, b1, W2, b2, Wg1, bg1, Wg2, bg2):
    src = edge_index[0].reshape(_NW, _NCHUNK, _CH)
    dst = edge_index[1].reshape(_NW, _NCHUNK, _CH)
    ep = jnp.pad(jnp.stack([src, dst], axis=2),
                 ((0, 0), (0, 1), (0, 0), (0, 0)))
    b1r = b1.reshape(1, _D)
    b2r = b2.reshape(1, _D)
    bg1r = bg1.reshape(1, _D)
    bg2r = bg2.reshape(1, _D)
    ones_deg = jnp.ones((_CH, _D), jnp.float32)
    zer_row = jnp.zeros((_RPT, _D), jnp.float32)

    degp = _deg_kernel(ep, ones_deg, zer_row)
    y1, dinv8 = _tc1(pose, degp, W1, b1r, W2, b2r, Wg1)
    accp1 = _scat_kernel(y1, ep, zer_row)
    y2 = _tc2(accp1, y1, dinv8, bg1r, Wg2)
    accp2 = _scat_kernel(y2, ep, zer_row)
    g = _tc3(accp2, y2, dinv8, bg2r)
    return g
